# baseline (speedup 1.0000x reference)
import functools

import jax
import jax.numpy as jnp
from jax import lax
from jax.experimental import pallas as pl
from jax.experimental.pallas import tpu as pltpu
from jax.experimental.pallas import tpu_sc as plsc

N_CTRL = 32


@functools.cache
def _make_kernel(B, D):
    rows_total = B * N_CTRL
    mesh = plsc.ScalarSubcoreMesh(axis_name="c", num_cores=1)

    @functools.partial(
        pl.kernel,
        mesh=mesh,
        out_type=jax.ShapeDtypeStruct((rows_total, D), jnp.float32),
        scratch_types=[
            pltpu.VMEM_SHARED((N_CTRL, D), jnp.float32),
            pltpu.SemaphoreType.DMA,
        ],
    )
    def seq_copy(table_hbm, out_hbm, spbuf, sem):
        pltpu.async_copy(table_hbm.at[pl.ds(0, N_CTRL), :], spbuf, sem).wait()
        copies = []
        for b in range(B):
            copies.append(pltpu.async_copy(
                spbuf, out_hbm.at[pl.ds(b * N_CTRL, N_CTRL), :], sem))
        for c in copies:
            c.wait()

    return seq_copy


def kernel(x, embed_table):
    B = x.shape[0]
    D = embed_table.shape[1]
    out_flat = _make_kernel(B, D)(embed_table)
    return out_flat.reshape(B, N_CTRL, D)


# speedup vs baseline: 1.0041x; 1.0041x over previous
import functools

import jax
import jax.numpy as jnp
from jax.experimental import pallas as pl
from jax.experimental.pallas import tpu as pltpu
from jax.experimental.pallas import tpu_sc as plsc

N_CTRL = 32


@functools.cache
def _make_kernel(B, D):
    rows_total = B * N_CTRL
    mesh = plsc.ScalarSubcoreMesh(axis_name="c", num_cores=1)

    @functools.partial(
        pl.kernel,
        mesh=mesh,
        out_type=jax.ShapeDtypeStruct((rows_total, D), jnp.float32),
        scratch_types=[
            pltpu.VMEM_SHARED((N_CTRL, D), jnp.float32),
            pltpu.SemaphoreType.DMA,
        ],
    )
    def seq_copy(table_hbm, out_hbm, spbuf, sem):
        pltpu.async_copy(table_hbm.at[pl.ds(0, N_CTRL), :], spbuf, sem).wait()
        copies = []
        for b in range(B):
            copies.append(pltpu.async_copy(
                spbuf, out_hbm.at[pl.ds(b * N_CTRL, N_CTRL), :], sem))
        for c in copies:
            c.wait()

    return seq_copy


def kernel(x, embed_table):
    B = x.shape[0]
    D = embed_table.shape[1]
    out_flat = _make_kernel(B, D)(embed_table)
    return out_flat.reshape(B, N_CTRL, D)
